# Initial kernel scaffold; baseline (speedup 1.0000x reference)
import jax
import jax.numpy as jnp
from jax.experimental import pallas as pl


def kernel(x, mask, h0, c0, W_ih, W_hh, b_ih, b_hh):
    raise NotImplementedError("write your pallas kernel here")


# speedup vs baseline: 1.8879x; 1.8879x over previous
import jax
import jax.numpy as jnp
from jax.experimental import pallas as pl

B, D, H = 8192, 512, 512
BLK = 256


def _lstm_block(x_ref, h_ref, c_ref, m_ref, wih_ref, whh_ref, b_ref,
                ho_ref, co_ref):
    gates = jnp.dot(x_ref[...], wih_ref[...], preferred_element_type=jnp.float32)
    gates = gates + jnp.dot(h_ref[...], whh_ref[...],
                            preferred_element_type=jnp.float32)
    gates = gates + b_ref[...]
    i = jax.nn.sigmoid(gates[:, 0 * H:1 * H])
    f = jax.nn.sigmoid(gates[:, 1 * H:2 * H])
    g = jnp.tanh(gates[:, 2 * H:3 * H])
    o = jax.nn.sigmoid(gates[:, 3 * H:4 * H])
    c_old = c_ref[...]
    c_new = f * c_old + i * g
    h_new = o * jnp.tanh(c_new)
    valid = m_ref[...] == 1
    ho_ref[...] = jnp.where(valid, h_new, h_ref[...])
    co_ref[...] = jnp.where(valid, c_new, c_old)


def kernel(x, mask, h0, c0, W_ih, W_hh, b_ih, b_hh):
    wih_t = W_ih.T
    whh_t = W_hh.T
    bias = (b_ih + b_hh)[None, :]
    mask2 = mask.reshape(B, 1)
    ho, co = pl.pallas_call(
        _lstm_block,
        grid=(B // BLK,),
        in_specs=[
            pl.BlockSpec((BLK, D), lambda i: (i, 0)),
            pl.BlockSpec((BLK, H), lambda i: (i, 0)),
            pl.BlockSpec((BLK, H), lambda i: (i, 0)),
            pl.BlockSpec((BLK, 1), lambda i: (i, 0)),
            pl.BlockSpec((D, 4 * H), lambda i: (0, 0)),
            pl.BlockSpec((H, 4 * H), lambda i: (0, 0)),
            pl.BlockSpec((1, 4 * H), lambda i: (0, 0)),
        ],
        out_specs=[
            pl.BlockSpec((BLK, H), lambda i: (i, 0)),
            pl.BlockSpec((BLK, H), lambda i: (i, 0)),
        ],
        out_shape=[
            jax.ShapeDtypeStruct((B, H), jnp.float32),
            jax.ShapeDtypeStruct((B, H), jnp.float32),
        ],
    )(x, h0, c0, mask2, wih_t, whh_t, bias)
    return ho, co


# speedup vs baseline: 1.9808x; 1.0492x over previous
import jax
import jax.numpy as jnp
from jax.experimental import pallas as pl

B, D, H = 8192, 512, 512
BLK = 256


def _lstm_block(x_ref, h_ref, c_ref, m_ref, wih_ref, whh_ref, b_ref,
                ho_ref, co_ref):
    gates = jnp.dot(x_ref[...].astype(jnp.bfloat16), wih_ref[...],
                    preferred_element_type=jnp.float32)
    gates = gates + jnp.dot(h_ref[...].astype(jnp.bfloat16), whh_ref[...],
                            preferred_element_type=jnp.float32)
    gates = gates + b_ref[...]
    i = jax.nn.sigmoid(gates[:, 0 * H:1 * H])
    f = jax.nn.sigmoid(gates[:, 1 * H:2 * H])
    g = jnp.tanh(gates[:, 2 * H:3 * H])
    o = jax.nn.sigmoid(gates[:, 3 * H:4 * H])
    c_old = c_ref[...]
    c_new = f * c_old + i * g
    h_new = o * jnp.tanh(c_new)
    valid = m_ref[...] == 1
    ho_ref[...] = jnp.where(valid, h_new, h_ref[...])
    co_ref[...] = jnp.where(valid, c_new, c_old)


def kernel(x, mask, h0, c0, W_ih, W_hh, b_ih, b_hh):
    wih_t = W_ih.T.astype(jnp.bfloat16)
    whh_t = W_hh.T.astype(jnp.bfloat16)
    bias = (b_ih + b_hh)[None, :]
    mask2 = mask.reshape(B, 1)
    ho, co = pl.pallas_call(
        _lstm_block,
        grid=(B // BLK,),
        in_specs=[
            pl.BlockSpec((BLK, D), lambda i: (i, 0)),
            pl.BlockSpec((BLK, H), lambda i: (i, 0)),
            pl.BlockSpec((BLK, H), lambda i: (i, 0)),
            pl.BlockSpec((BLK, 1), lambda i: (i, 0)),
            pl.BlockSpec((D, 4 * H), lambda i: (0, 0)),
            pl.BlockSpec((H, 4 * H), lambda i: (0, 0)),
            pl.BlockSpec((1, 4 * H), lambda i: (0, 0)),
        ],
        out_specs=[
            pl.BlockSpec((BLK, H), lambda i: (i, 0)),
            pl.BlockSpec((BLK, H), lambda i: (i, 0)),
        ],
        out_shape=[
            jax.ShapeDtypeStruct((B, H), jnp.float32),
            jax.ShapeDtypeStruct((B, H), jnp.float32),
        ],
    )(x, h0, c0, mask2, wih_t, whh_t, bias)
    return ho, co


# speedup vs baseline: 2.0992x; 1.0597x over previous
import jax
import jax.numpy as jnp
from jax.experimental import pallas as pl

B, D, H = 8192, 512, 512
BLK = 512


def _lstm_block(x_ref, h_ref, c_ref, m_ref, wih_ref, whh_ref, b_ref,
                ho_ref, co_ref):
    gates = jnp.dot(x_ref[...].astype(jnp.bfloat16), wih_ref[...],
                    preferred_element_type=jnp.float32)
    gates = gates + jnp.dot(h_ref[...].astype(jnp.bfloat16), whh_ref[...],
                            preferred_element_type=jnp.float32)
    gates = gates + b_ref[...]
    i = jax.nn.sigmoid(gates[:, 0 * H:1 * H])
    f = jax.nn.sigmoid(gates[:, 1 * H:2 * H])
    g = jnp.tanh(gates[:, 2 * H:3 * H])
    o = jax.nn.sigmoid(gates[:, 3 * H:4 * H])
    c_old = c_ref[...]
    c_new = f * c_old + i * g
    h_new = o * jnp.tanh(c_new)
    valid = m_ref[...] == 1
    ho_ref[...] = jnp.where(valid, h_new, h_ref[...])
    co_ref[...] = jnp.where(valid, c_new, c_old)


def kernel(x, mask, h0, c0, W_ih, W_hh, b_ih, b_hh):
    wih_t = W_ih.T.astype(jnp.bfloat16)
    whh_t = W_hh.T.astype(jnp.bfloat16)
    bias = (b_ih + b_hh)[None, :]
    mask2 = mask.reshape(B, 1)
    ho, co = pl.pallas_call(
        _lstm_block,
        grid=(B // BLK,),
        in_specs=[
            pl.BlockSpec((BLK, D), lambda i: (i, 0)),
            pl.BlockSpec((BLK, H), lambda i: (i, 0)),
            pl.BlockSpec((BLK, H), lambda i: (i, 0)),
            pl.BlockSpec((BLK, 1), lambda i: (i, 0)),
            pl.BlockSpec((D, 4 * H), lambda i: (0, 0)),
            pl.BlockSpec((H, 4 * H), lambda i: (0, 0)),
            pl.BlockSpec((1, 4 * H), lambda i: (0, 0)),
        ],
        out_specs=[
            pl.BlockSpec((BLK, H), lambda i: (i, 0)),
            pl.BlockSpec((BLK, H), lambda i: (i, 0)),
        ],
        out_shape=[
            jax.ShapeDtypeStruct((B, H), jnp.float32),
            jax.ShapeDtypeStruct((B, H), jnp.float32),
        ],
    )(x, h0, c0, mask2, wih_t, whh_t, bias)
    return ho, co


# speedup vs baseline: 2.1034x; 1.0020x over previous
import jax
import jax.numpy as jnp
from jax.experimental import pallas as pl
from jax.experimental.pallas import tpu as pltpu

B, D, H = 8192, 512, 512
BLK = 512


def _lstm_block(x_ref, h_ref, c_ref, m_ref, wih_ref, whh_ref, b_ref,
                ho_ref, co_ref):
    gates = jnp.dot(x_ref[...].astype(jnp.bfloat16), wih_ref[...],
                    preferred_element_type=jnp.float32)
    gates = gates + jnp.dot(h_ref[...].astype(jnp.bfloat16), whh_ref[...],
                            preferred_element_type=jnp.float32)
    gates = gates + b_ref[...]
    i = jax.nn.sigmoid(gates[:, 0 * H:1 * H])
    f = jax.nn.sigmoid(gates[:, 1 * H:2 * H])
    g = jnp.tanh(gates[:, 2 * H:3 * H])
    o = jax.nn.sigmoid(gates[:, 3 * H:4 * H])
    c_old = c_ref[...]
    c_new = f * c_old + i * g
    h_new = o * jnp.tanh(c_new)
    valid = m_ref[...] == 1
    ho_ref[...] = jnp.where(valid, h_new, h_ref[...])
    co_ref[...] = jnp.where(valid, c_new, c_old)


def kernel(x, mask, h0, c0, W_ih, W_hh, b_ih, b_hh):
    wih_t = W_ih.T.astype(jnp.bfloat16)
    whh_t = W_hh.T.astype(jnp.bfloat16)
    bias = (b_ih + b_hh)[None, :]
    mask2 = mask.reshape(B, 1)
    ho, co = pl.pallas_call(
        _lstm_block,
        grid=(B // BLK,),
        in_specs=[
            pl.BlockSpec((BLK, D), lambda i: (i, 0)),
            pl.BlockSpec((BLK, H), lambda i: (i, 0)),
            pl.BlockSpec((BLK, H), lambda i: (i, 0)),
            pl.BlockSpec((BLK, 1), lambda i: (i, 0)),
            pl.BlockSpec((D, 4 * H), lambda i: (0, 0)),
            pl.BlockSpec((H, 4 * H), lambda i: (0, 0)),
            pl.BlockSpec((1, 4 * H), lambda i: (0, 0)),
        ],
        out_specs=[
            pl.BlockSpec((BLK, H), lambda i: (i, 0)),
            pl.BlockSpec((BLK, H), lambda i: (i, 0)),
        ],
        out_shape=[
            jax.ShapeDtypeStruct((B, H), jnp.float32),
            jax.ShapeDtypeStruct((B, H), jnp.float32),
        ],
        compiler_params=pltpu.CompilerParams(
            dimension_semantics=("parallel",),
        ),
    )(x, h0, c0, mask2, wih_t, whh_t, bias)
    return ho, co


# speedup vs baseline: 2.3213x; 1.1036x over previous
import jax
import jax.numpy as jnp
from jax.experimental import pallas as pl
from jax.experimental.pallas import tpu as pltpu

B, D, H = 8192, 512, 512
BLK = 512


def _lstm_block(x_ref, h_ref, c_ref, m_ref, wih_ref, whh_ref, b_ref,
                ho_ref, co_ref):
    dn = (((1,), (1,)), ((), ()))
    gates = jax.lax.dot_general(x_ref[...], wih_ref[...], dn,
                                preferred_element_type=jnp.float32)
    gates = gates + jax.lax.dot_general(h_ref[...], whh_ref[...], dn,
                                        preferred_element_type=jnp.float32)
    gates = gates + b_ref[...]
    i = jax.nn.sigmoid(gates[:, 0 * H:1 * H])
    f = jax.nn.sigmoid(gates[:, 1 * H:2 * H])
    g = jnp.tanh(gates[:, 2 * H:3 * H])
    o = jax.nn.sigmoid(gates[:, 3 * H:4 * H])
    c_old = c_ref[...]
    c_new = f * c_old + i * g
    h_new = o * jnp.tanh(c_new)
    valid = m_ref[...] == 1
    ho_ref[...] = jnp.where(valid, h_new, h_ref[...])
    co_ref[...] = jnp.where(valid, c_new, c_old)


def kernel(x, mask, h0, c0, W_ih, W_hh, b_ih, b_hh):
    bias = (b_ih + b_hh)[None, :]
    mask2 = mask.reshape(B, 1)
    ho, co = pl.pallas_call(
        _lstm_block,
        grid=(B // BLK,),
        in_specs=[
            pl.BlockSpec((BLK, D), lambda i: (i, 0)),
            pl.BlockSpec((BLK, H), lambda i: (i, 0)),
            pl.BlockSpec((BLK, H), lambda i: (i, 0)),
            pl.BlockSpec((BLK, 1), lambda i: (i, 0)),
            pl.BlockSpec((4 * H, D), lambda i: (0, 0)),
            pl.BlockSpec((4 * H, H), lambda i: (0, 0)),
            pl.BlockSpec((1, 4 * H), lambda i: (0, 0)),
        ],
        out_specs=[
            pl.BlockSpec((BLK, H), lambda i: (i, 0)),
            pl.BlockSpec((BLK, H), lambda i: (i, 0)),
        ],
        out_shape=[
            jax.ShapeDtypeStruct((B, H), jnp.float32),
            jax.ShapeDtypeStruct((B, H), jnp.float32),
        ],
        compiler_params=pltpu.CompilerParams(
            dimension_semantics=("parallel",),
        ),
    )(x, h0, c0, mask2, W_ih, W_hh, bias)
    return ho, co


# speedup vs baseline: 2.3875x; 1.0285x over previous
import jax
import jax.numpy as jnp
from jax.experimental import pallas as pl
from jax.experimental.pallas import tpu as pltpu

B, D, H = 8192, 512, 512
BLK = 512


def _lstm_block(x_ref, h_ref, c_ref, m_ref, wih_ref, whh_ref,
                bih_ref, bhh_ref, ho_ref, co_ref):
    dn = (((1,), (1,)), ((), ()))
    gates = jax.lax.dot_general(x_ref[...], wih_ref[...], dn,
                                preferred_element_type=jnp.float32)
    gates = gates + jax.lax.dot_general(h_ref[...], whh_ref[...], dn,
                                        preferred_element_type=jnp.float32)
    gates = gates + (bih_ref[...] + bhh_ref[...])
    i = jax.nn.sigmoid(gates[:, 0 * H:1 * H])
    f = jax.nn.sigmoid(gates[:, 1 * H:2 * H])
    g = jnp.tanh(gates[:, 2 * H:3 * H])
    o = jax.nn.sigmoid(gates[:, 3 * H:4 * H])
    c_old = c_ref[...]
    c_new = f * c_old + i * g
    h_new = o * jnp.tanh(c_new)
    valid = m_ref[...] == 1
    ho_ref[...] = jnp.where(valid, h_new, h_ref[...])
    co_ref[...] = jnp.where(valid, c_new, c_old)


def kernel(x, mask, h0, c0, W_ih, W_hh, b_ih, b_hh):
    mask2 = mask.reshape(B, 1)
    bih2 = b_ih.reshape(1, 4 * H)
    bhh2 = b_hh.reshape(1, 4 * H)
    ho, co = pl.pallas_call(
        _lstm_block,
        grid=(B // BLK,),
        in_specs=[
            pl.BlockSpec((BLK, D), lambda i: (i, 0)),
            pl.BlockSpec((BLK, H), lambda i: (i, 0)),
            pl.BlockSpec((BLK, H), lambda i: (i, 0)),
            pl.BlockSpec((BLK, 1), lambda i: (i, 0)),
            pl.BlockSpec((4 * H, D), lambda i: (0, 0)),
            pl.BlockSpec((4 * H, H), lambda i: (0, 0)),
            pl.BlockSpec((1, 4 * H), lambda i: (0, 0)),
            pl.BlockSpec((1, 4 * H), lambda i: (0, 0)),
        ],
        out_specs=[
            pl.BlockSpec((BLK, H), lambda i: (i, 0)),
            pl.BlockSpec((BLK, H), lambda i: (i, 0)),
        ],
        out_shape=[
            jax.ShapeDtypeStruct((B, H), jnp.float32),
            jax.ShapeDtypeStruct((B, H), jnp.float32),
        ],
        compiler_params=pltpu.CompilerParams(
            dimension_semantics=("parallel",),
        ),
    )(x, h0, c0, mask2, W_ih, W_hh, bih2, bhh2)
    return ho, co


# speedup vs baseline: 2.5202x; 1.0556x over previous
import jax
import jax.numpy as jnp
from jax.experimental import pallas as pl
from jax.experimental.pallas import tpu as pltpu

B, D, H = 8192, 512, 512
BLK = 1024


def _lstm_block(x_ref, h_ref, c_ref, m_ref, wih_ref, whh_ref,
                bih_ref, bhh_ref, ho_ref, co_ref):
    dn = (((1,), (1,)), ((), ()))
    gates = jax.lax.dot_general(x_ref[...], wih_ref[...], dn,
                                preferred_element_type=jnp.float32)
    gates = gates + jax.lax.dot_general(h_ref[...], whh_ref[...], dn,
                                        preferred_element_type=jnp.float32)
    gates = gates + (bih_ref[...] + bhh_ref[...])
    i = jax.nn.sigmoid(gates[:, 0 * H:1 * H])
    f = jax.nn.sigmoid(gates[:, 1 * H:2 * H])
    g = jnp.tanh(gates[:, 2 * H:3 * H])
    o = jax.nn.sigmoid(gates[:, 3 * H:4 * H])
    c_old = c_ref[...]
    c_new = f * c_old + i * g
    h_new = o * jnp.tanh(c_new)
    valid = m_ref[...] == 1
    ho_ref[...] = jnp.where(valid, h_new, h_ref[...])
    co_ref[...] = jnp.where(valid, c_new, c_old)


def kernel(x, mask, h0, c0, W_ih, W_hh, b_ih, b_hh):
    mask2 = mask.reshape(B, 1)
    bih2 = b_ih.reshape(1, 4 * H)
    bhh2 = b_hh.reshape(1, 4 * H)
    ho, co = pl.pallas_call(
        _lstm_block,
        grid=(B // BLK,),
        in_specs=[
            pl.BlockSpec((BLK, D), lambda i: (i, 0)),
            pl.BlockSpec((BLK, H), lambda i: (i, 0)),
            pl.BlockSpec((BLK, H), lambda i: (i, 0)),
            pl.BlockSpec((BLK, 1), lambda i: (i, 0)),
            pl.BlockSpec((4 * H, D), lambda i: (0, 0)),
            pl.BlockSpec((4 * H, H), lambda i: (0, 0)),
            pl.BlockSpec((1, 4 * H), lambda i: (0, 0)),
            pl.BlockSpec((1, 4 * H), lambda i: (0, 0)),
        ],
        out_specs=[
            pl.BlockSpec((BLK, H), lambda i: (i, 0)),
            pl.BlockSpec((BLK, H), lambda i: (i, 0)),
        ],
        out_shape=[
            jax.ShapeDtypeStruct((B, H), jnp.float32),
            jax.ShapeDtypeStruct((B, H), jnp.float32),
        ],
        compiler_params=pltpu.CompilerParams(
            dimension_semantics=("parallel",),
        ),
    )(x, h0, c0, mask2, W_ih, W_hh, bih2, bhh2)
    return ho, co
